# Initial kernel scaffold; baseline (speedup 1.0000x reference)
import jax
import jax.numpy as jnp
from jax.experimental import pallas as pl


def kernel(xyz, xyz_batch_cnt, point_features, pillar_indices, pillar_set_indices, point_set_indices, W1, gamma1, beta1):
    raise NotImplementedError("write your pallas kernel here")


# speedup vs baseline: 25.8485x; 25.8485x over previous
import functools

import jax
import jax.numpy as jnp
from jax import lax
from jax.experimental import pallas as pl
from jax.experimental.pallas import tpu as pltpu
from jax.experimental.pallas import tpu_sc as plsc

NW = 32
C_OUT = 64
CHUNK = 128
NEG = float("-inf")


def _mlp_body(g_ref, wt_ref, gamma_ref, a_ref):
    a = lax.dot_general(g_ref[...], wt_ref[...], (((1,), (0,)), ((), ())),
                        precision=lax.Precision.HIGHEST,
                        preferred_element_type=jnp.float32)
    a_ref[...] = a * gamma_ref[...]


def _run_mlp(g, w1t, gamma_row):
    n = g.shape[0]
    bs = 2000
    return pl.pallas_call(
        _mlp_body,
        grid=(n // bs,),
        in_specs=[
            pl.BlockSpec((bs, 32), lambda i: (i, 0)),
            pl.BlockSpec((32, C_OUT), lambda i: (0, 0)),
            pl.BlockSpec((1, C_OUT), lambda i: (0, 0)),
        ],
        out_specs=pl.BlockSpec((bs, C_OUT), lambda i: (i, 0)),
        out_shape=jax.ShapeDtypeStruct((n, C_OUT), jnp.float32),
    )(g, w1t, gamma_row)


def _make_segmax(n_pts, l_pad, mw, m_pad):
    mesh = plsc.VectorSubcoreMesh(core_axis_name="c", subcore_axis_name="s")

    @functools.partial(
        pl.kernel,
        out_type=jax.ShapeDtypeStruct((m_pad, C_OUT), jnp.float32),
        mesh=mesh,
        compiler_params=pltpu.CompilerParams(use_tc_tiling_on_sc=False),
        scratch_types=[
            pltpu.VMEM((mw + 1, C_OUT), jnp.float32),
            pltpu.VMEM((CHUNK,), jnp.int32),
            pltpu.VMEM((CHUNK,), jnp.int32),
            pltpu.VMEM((CHUNK, C_OUT), jnp.float32),
            pltpu.VMEM((48,), jnp.int32),
            pltpu.SemaphoreType.DMA,
        ],
    )
    def segmax(a_hbm, pidx_hbm, psi_hbm, bounds_hbm, s_hbm,
               acc, pidx_v, psi_v, rows_v, bounds_v, sem):
        wid = lax.axis_index("s") * 2 + lax.axis_index("c")
        m0 = wid * mw

        neg = jnp.full((16,), NEG, jnp.float32)

        def init_row(r, carry):
            for c in range(4):
                acc[r, pl.ds(c * 16, 16)] = neg
            return carry
        lax.fori_loop(0, mw + 1, init_row, 0, unroll=False)

        pltpu.sync_copy(bounds_hbm, bounds_v)
        bv = bounds_v[pl.ds(wid, 16)]
        lo = bv[0]
        hi = bv[1]
        lo8 = lo & jnp.int32(-8)
        nchunks = (hi - lo8 + (CHUNK - 1)) >> 7

        def chunk_body(t, carry):
            base = pl.multiple_of(lo8 + t * CHUNK, 8)
            pltpu.sync_copy(pidx_hbm.at[pl.ds(base, CHUNK)], pidx_v)
            pltpu.sync_copy(psi_hbm.at[pl.ds(base, CHUNK)], psi_v)
            pltpu.async_copy(a_hbm.at[pidx_v], rows_v, sem).wait()

            def group(gi, c2):
                sv = psi_v[pl.ds(gi * 16, 16)] - m0
                for j in range(16):
                    r = sv[j]
                    r = jnp.where((r < 0) | (r >= mw), mw, r)
                    i = gi * 16 + j
                    for c in range(4):
                        sl = pl.ds(c * 16, 16)
                        acc[r, sl] = jnp.maximum(acc[r, sl], rows_v[i, sl])
                return c2
            lax.fori_loop(0, CHUNK // 16, group, 0, unroll=False)
            return carry
        lax.fori_loop(0, nchunks, chunk_body, 0, unroll=False)

        pltpu.sync_copy(acc.at[pl.ds(0, mw)], s_hbm.at[pl.ds(m0, mw)])

    return segmax


def _epi_body(s_ref, pif_ref, wxyz_ref, gamma_ref, beta_ref, o_ref):
    pif = pif_ref[...]
    cx = (pif[:, 2:3] + 0.5) * 0.2 - 51.2
    cy = (pif[:, 1:2] + 0.5) * 0.2 - 51.2
    cz = jnp.float32(-1.0)
    b = cx * wxyz_ref[0:1, :] + cy * wxyz_ref[1:2, :] + cz * wxyz_ref[2:3, :]
    b = b * gamma_ref[...] - beta_ref[...]
    o_ref[...] = jnp.maximum(s_ref[...] - b, 0.0)


def _run_epilogue(s, pif32, wxyz, gamma_row, beta_row, mw):
    m_pad = s.shape[0]
    return pl.pallas_call(
        _epi_body,
        grid=(m_pad // mw,),
        in_specs=[
            pl.BlockSpec((mw, C_OUT), lambda i: (i, 0)),
            pl.BlockSpec((mw, 3), lambda i: (i, 0)),
            pl.BlockSpec((8, C_OUT), lambda i: (0, 0)),
            pl.BlockSpec((1, C_OUT), lambda i: (0, 0)),
            pl.BlockSpec((1, C_OUT), lambda i: (0, 0)),
        ],
        out_specs=pl.BlockSpec((mw, C_OUT), lambda i: (i, 0)),
        out_shape=jax.ShapeDtypeStruct((m_pad, C_OUT), jnp.float32),
    )(s, pif32, wxyz, gamma_row, beta_row)


def kernel(xyz, xyz_batch_cnt, point_features, pillar_indices,
           pillar_set_indices, point_set_indices, W1, gamma1, beta1):
    n = point_features.shape[0]
    m = pillar_indices.shape[0]
    l = pillar_set_indices.shape[0]
    mw = (-(-m // NW) + 7) // 8 * 8
    m_pad = NW * mw
    l_pad = -(-(l + 256) // CHUNK) * CHUNK

    w1t = W1.T
    gamma_row = gamma1.reshape(1, C_OUT)
    beta_row = beta1.reshape(1, C_OUT)

    g = jnp.concatenate([point_features, xyz], axis=1)
    a = _run_mlp(g, w1t, gamma_row)

    thresholds = jnp.arange(NW + 1, dtype=jnp.int32) * mw
    bounds = jnp.searchsorted(pillar_set_indices, thresholds,
                              side="left").astype(jnp.int32)
    bounds_pad = jnp.zeros((48,), jnp.int32).at[: NW + 1].set(bounds)

    psi_pad = jnp.full((l_pad,), jnp.int32(1 << 29), jnp.int32)
    psi_pad = psi_pad.at[:l].set(pillar_set_indices)
    pidx_pad = jnp.zeros((l_pad,), jnp.int32).at[:l].set(point_set_indices)

    s = _make_segmax(n, l_pad, mw, m_pad)(a, pidx_pad, psi_pad, bounds_pad)

    pif32 = jnp.zeros((m_pad, 3), jnp.float32)
    pif32 = pif32.at[:m].set(pillar_indices.astype(jnp.float32))
    wxyz = jnp.zeros((8, C_OUT), jnp.float32).at[:3].set(w1t[29:32])
    out = _run_epilogue(s, pif32, wxyz, gamma_row, beta_row, mw)
    return out[:m]


# speedup vs baseline: 32.1198x; 1.2426x over previous
import functools

import jax
import jax.numpy as jnp
from jax import lax
from jax.experimental import pallas as pl
from jax.experimental.pallas import tpu as pltpu
from jax.experimental.pallas import tpu_sc as plsc

NW = 32
C_OUT = 64
CHUNK = 128
NEG = float("-inf")


def _mlp_body(g_ref, wt_ref, gamma_ref, a_ref):
    a = lax.dot_general(g_ref[...], wt_ref[...], (((1,), (0,)), ((), ())),
                        precision=lax.Precision.HIGHEST,
                        preferred_element_type=jnp.float32)
    a_ref[...] = a * gamma_ref[...]


def _run_mlp(g, w1t, gamma_row):
    n = g.shape[0]
    bs = 2000
    return pl.pallas_call(
        _mlp_body,
        grid=(n // bs,),
        in_specs=[
            pl.BlockSpec((bs, 32), lambda i: (i, 0)),
            pl.BlockSpec((32, C_OUT), lambda i: (0, 0)),
            pl.BlockSpec((1, C_OUT), lambda i: (0, 0)),
        ],
        out_specs=pl.BlockSpec((bs, C_OUT), lambda i: (i, 0)),
        out_shape=jax.ShapeDtypeStruct((n, C_OUT), jnp.float32),
    )(g, w1t, gamma_row)


def _make_segmax(n_pts, l_pad, mw, m_pad):
    mesh = plsc.VectorSubcoreMesh(core_axis_name="c", subcore_axis_name="s")

    @functools.partial(
        pl.kernel,
        out_type=jax.ShapeDtypeStruct((m_pad, C_OUT), jnp.float32),
        mesh=mesh,
        compiler_params=pltpu.CompilerParams(use_tc_tiling_on_sc=False),
        scratch_types=[
            pltpu.VMEM((mw + 1, C_OUT), jnp.float32),
            pltpu.VMEM((CHUNK,), jnp.int32),
            pltpu.VMEM((CHUNK,), jnp.int32),
            pltpu.VMEM((CHUNK, C_OUT), jnp.float32),
            pltpu.VMEM((48,), jnp.int32),
            pltpu.SemaphoreType.DMA,
        ],
    )
    def segmax(a_hbm, pidx_hbm, psi_hbm, bounds_hbm, s_hbm,
               acc, pidx_v, psi_v, rows_v, bounds_v, sem):
        wid = lax.axis_index("s") * 2 + lax.axis_index("c")
        m0 = wid * mw

        neg = jnp.full((16,), NEG, jnp.float32)

        def init_row(r, carry):
            for c in range(4):
                acc[r, pl.ds(c * 16, 16)] = neg
            return carry
        lax.fori_loop(0, mw + 1, init_row, 0, unroll=False)

        pltpu.sync_copy(bounds_hbm, bounds_v)
        bv = bounds_v[pl.ds(wid, 16)]
        lo = bv[0]
        hi = bv[1]
        lo8 = lo & jnp.int32(-8)
        nchunks = (hi - lo8 + (CHUNK - 1)) >> 7

        def chunk_body(t, carry):
            base = pl.multiple_of(lo8 + t * CHUNK, 8)
            pltpu.sync_copy(pidx_hbm.at[pl.ds(base, CHUNK)], pidx_v)
            pltpu.sync_copy(psi_hbm.at[pl.ds(base, CHUNK)], psi_v)
            pltpu.async_copy(a_hbm.at[pidx_v], rows_v, sem).wait()

            def group(gi, c2):
                rp, m_0, m_1, m_2, m_3 = c2
                sv = psi_v[pl.ds(gi * 16, 16)] - m0
                rv = jnp.where((sv < 0) | (sv >= mw), mw, sv)
                for j in range(16):
                    r = rv[j]
                    i = gi * 16 + j
                    row0 = rows_v[i, pl.ds(0, 16)]
                    row1 = rows_v[i, pl.ds(16, 16)]
                    row2 = rows_v[i, pl.ds(32, 16)]
                    row3 = rows_v[i, pl.ds(48, 16)]
                    new = r != rp
                    m_0 = jnp.maximum(jnp.where(new, neg16, m_0), row0)
                    m_1 = jnp.maximum(jnp.where(new, neg16, m_1), row1)
                    m_2 = jnp.maximum(jnp.where(new, neg16, m_2), row2)
                    m_3 = jnp.maximum(jnp.where(new, neg16, m_3), row3)
                    acc[r, pl.ds(0, 16)] = m_0
                    acc[r, pl.ds(16, 16)] = m_1
                    acc[r, pl.ds(32, 16)] = m_2
                    acc[r, pl.ds(48, 16)] = m_3
                    rp = r
                return rp, m_0, m_1, m_2, m_3
            return lax.fori_loop(0, CHUNK // 16, group, carry, unroll=False)

        neg16 = jnp.full((16,), NEG, jnp.float32)
        carry0 = (jnp.int32(mw), neg16, neg16, neg16, neg16)
        lax.fori_loop(0, nchunks, chunk_body, carry0, unroll=False)

        pltpu.sync_copy(acc.at[pl.ds(0, mw)], s_hbm.at[pl.ds(m0, mw)])

    return segmax


def _epi_body(s_ref, pif_ref, wxyz_ref, gamma_ref, beta_ref, o_ref):
    pif = pif_ref[...]
    cx = (pif[:, 2:3] + 0.5) * 0.2 - 51.2
    cy = (pif[:, 1:2] + 0.5) * 0.2 - 51.2
    cz = jnp.float32(-1.0)
    b = cx * wxyz_ref[0:1, :] + cy * wxyz_ref[1:2, :] + cz * wxyz_ref[2:3, :]
    b = b * gamma_ref[...] - beta_ref[...]
    o_ref[...] = jnp.maximum(s_ref[...] - b, 0.0)


def _run_epilogue(s, pif32, wxyz, gamma_row, beta_row, mw):
    m_pad = s.shape[0]
    return pl.pallas_call(
        _epi_body,
        grid=(m_pad // mw,),
        in_specs=[
            pl.BlockSpec((mw, C_OUT), lambda i: (i, 0)),
            pl.BlockSpec((mw, 3), lambda i: (i, 0)),
            pl.BlockSpec((8, C_OUT), lambda i: (0, 0)),
            pl.BlockSpec((1, C_OUT), lambda i: (0, 0)),
            pl.BlockSpec((1, C_OUT), lambda i: (0, 0)),
        ],
        out_specs=pl.BlockSpec((mw, C_OUT), lambda i: (i, 0)),
        out_shape=jax.ShapeDtypeStruct((m_pad, C_OUT), jnp.float32),
    )(s, pif32, wxyz, gamma_row, beta_row)


def kernel(xyz, xyz_batch_cnt, point_features, pillar_indices,
           pillar_set_indices, point_set_indices, W1, gamma1, beta1):
    n = point_features.shape[0]
    m = pillar_indices.shape[0]
    l = pillar_set_indices.shape[0]
    mw = (-(-m // NW) + 7) // 8 * 8
    m_pad = NW * mw
    l_pad = -(-(l + 256) // CHUNK) * CHUNK

    w1t = W1.T
    gamma_row = gamma1.reshape(1, C_OUT)
    beta_row = beta1.reshape(1, C_OUT)

    g = jnp.concatenate([point_features, xyz], axis=1)
    a = _run_mlp(g, w1t, gamma_row)

    thresholds = jnp.arange(NW + 1, dtype=jnp.int32) * mw
    bounds = jnp.searchsorted(pillar_set_indices, thresholds,
                              side="left").astype(jnp.int32)
    bounds_pad = jnp.zeros((48,), jnp.int32).at[: NW + 1].set(bounds)

    psi_pad = jnp.full((l_pad,), jnp.int32(1 << 29), jnp.int32)
    psi_pad = psi_pad.at[:l].set(pillar_set_indices)
    pidx_pad = jnp.zeros((l_pad,), jnp.int32).at[:l].set(point_set_indices)

    s = _make_segmax(n, l_pad, mw, m_pad)(a, pidx_pad, psi_pad, bounds_pad)

    pif32 = jnp.zeros((m_pad, 3), jnp.float32)
    pif32 = pif32.at[:m].set(pillar_indices.astype(jnp.float32))
    wxyz = jnp.zeros((8, C_OUT), jnp.float32).at[:3].set(w1t[29:32])
    out = _run_epilogue(s, pif32, wxyz, gamma_row, beta_row, mw)
    return out[:m]


# speedup vs baseline: 43.9841x; 1.3694x over previous
import functools

import jax
import jax.numpy as jnp
from jax import lax
from jax.experimental import pallas as pl
from jax.experimental.pallas import tpu as pltpu
from jax.experimental.pallas import tpu_sc as plsc

NW = 32
C_OUT = 64
CHUNK = 128
SUP = 2048
SUP_LOG2 = 11
NCH = SUP // CHUNK
NEG = float("-inf")


def _mlp_body(g_ref, wt_ref, gamma_ref, a_ref):
    a = lax.dot_general(g_ref[...], wt_ref[...], (((1,), (0,)), ((), ())),
                        precision=lax.Precision.HIGHEST,
                        preferred_element_type=jnp.float32)
    a_ref[...] = a * gamma_ref[...]


def _run_mlp(g, w1t, gamma_row):
    n = g.shape[0]
    bs = 2000
    return pl.pallas_call(
        _mlp_body,
        grid=(n // bs,),
        in_specs=[
            pl.BlockSpec((bs, 32), lambda i: (i, 0)),
            pl.BlockSpec((32, C_OUT), lambda i: (0, 0)),
            pl.BlockSpec((1, C_OUT), lambda i: (0, 0)),
        ],
        out_specs=pl.BlockSpec((bs, C_OUT), lambda i: (i, 0)),
        out_shape=jax.ShapeDtypeStruct((n, C_OUT), jnp.float32),
    )(g, w1t, gamma_row)


def _make_segmax(n_pts, l_pad, mw, m_pad):
    mesh = plsc.VectorSubcoreMesh(core_axis_name="c", subcore_axis_name="s")

    @functools.partial(
        pl.kernel,
        out_type=jax.ShapeDtypeStruct((m_pad, C_OUT), jnp.float32),
        mesh=mesh,
        compiler_params=pltpu.CompilerParams(use_tc_tiling_on_sc=False),
        scratch_types=[
            pltpu.VMEM((mw + 1, C_OUT), jnp.float32),
            pltpu.VMEM((SUP,), jnp.int32),
            pltpu.VMEM((SUP,), jnp.int32),
            pltpu.VMEM((2, CHUNK, C_OUT), jnp.float32),
            pltpu.VMEM((48,), jnp.int32),
            pltpu.SemaphoreType.DMA,
            pltpu.SemaphoreType.DMA,
            pltpu.SemaphoreType.DMA,
        ],
    )
    def segmax(a_hbm, pidx_hbm, psi_hbm, bounds_hbm, s_hbm,
               acc, pidx_v, psi_v, rows_v, bounds_v, sem_i, sem_g0, sem_g1):
        wid = lax.axis_index("s") * 2 + lax.axis_index("c")
        m0 = wid * mw
        sem_g = (sem_g0, sem_g1)

        neg16 = jnp.full((16,), NEG, jnp.float32)

        def init_row(r, carry):
            for c in range(4):
                acc[r, pl.ds(c * 16, 16)] = neg16
            return carry
        lax.fori_loop(0, mw + 1, init_row, 0, unroll=False)

        pltpu.sync_copy(bounds_hbm, bounds_v)
        bv = bounds_v[pl.ds(wid, 16)]
        lo = bv[0]
        hi = bv[1]
        lo8 = lo & jnp.int32(-8)
        nsup = jnp.maximum((hi - lo8 + (SUP - 1)) >> SUP_LOG2, 1)

        def issue_ids(s):
            base = pl.multiple_of(lo8 + s * SUP, 8)
            pltpu.async_copy(pidx_hbm.at[pl.ds(base, SUP)], pidx_v, sem_i)
            pltpu.async_copy(psi_hbm.at[pl.ds(base, SUP)], psi_v, sem_i)

        def wait_ids():
            pltpu.make_async_copy(pidx_hbm.at[pl.ds(0, SUP)], pidx_v, sem_i).wait()
            pltpu.make_async_copy(psi_hbm.at[pl.ds(0, SUP)], psi_v, sem_i).wait()

        def issue_gather(t, b):
            pltpu.async_copy(a_hbm.at[pidx_v.at[pl.ds(t * CHUNK, CHUNK)]],
                             rows_v.at[b], sem_g[b])

        def wait_gather(b):
            pltpu.make_async_copy(a_hbm.at[pidx_v.at[pl.ds(0, CHUNK)]],
                                  rows_v.at[b], sem_g[b]).wait()

        def compute_chunk(t, b, carry):
            def group(gi, c2):
                rp, m_0, m_1, m_2, m_3 = c2
                sv = psi_v[pl.ds(t * CHUNK + gi * 16, 16)] - m0
                rv = jnp.where((sv < 0) | (sv >= mw), mw, sv)
                for j in range(16):
                    r = rv[j]
                    i = gi * 16 + j
                    row0 = rows_v[b, i, pl.ds(0, 16)]
                    row1 = rows_v[b, i, pl.ds(16, 16)]
                    row2 = rows_v[b, i, pl.ds(32, 16)]
                    row3 = rows_v[b, i, pl.ds(48, 16)]
                    new = r != rp
                    m_0 = jnp.maximum(jnp.where(new, neg16, m_0), row0)
                    m_1 = jnp.maximum(jnp.where(new, neg16, m_1), row1)
                    m_2 = jnp.maximum(jnp.where(new, neg16, m_2), row2)
                    m_3 = jnp.maximum(jnp.where(new, neg16, m_3), row3)
                    acc[r, pl.ds(0, 16)] = m_0
                    acc[r, pl.ds(16, 16)] = m_1
                    acc[r, pl.ds(32, 16)] = m_2
                    acc[r, pl.ds(48, 16)] = m_3
                    rp = r
                return rp, m_0, m_1, m_2, m_3
            return lax.fori_loop(0, CHUNK // 16, group, carry, unroll=False)

        issue_ids(jnp.int32(0))

        def sup_body(s, carry):
            wait_ids()
            issue_gather(0, 0)
            for t in range(NCH):
                if t + 1 < NCH:
                    issue_gather(t + 1, (t + 1) % 2)
                wait_gather(t % 2)
                carry2 = compute_chunk(t, t % 2, carry if t == 0 else carry2)
                carry = carry2

            @pl.when(s + 1 < nsup)
            def _():
                issue_ids(s + 1)
            return carry

        carry0 = (jnp.int32(mw), neg16, neg16, neg16, neg16)
        lax.fori_loop(0, nsup, sup_body, carry0, unroll=False)

        pltpu.sync_copy(acc.at[pl.ds(0, mw)], s_hbm.at[pl.ds(m0, mw)])

    return segmax


def _epi_body(s_ref, pif_ref, wxyz_ref, gamma_ref, beta_ref, o_ref):
    pif = pif_ref[...]
    cx = (pif[:, 2:3] + 0.5) * 0.2 - 51.2
    cy = (pif[:, 1:2] + 0.5) * 0.2 - 51.2
    cz = jnp.float32(-1.0)
    b = cx * wxyz_ref[0:1, :] + cy * wxyz_ref[1:2, :] + cz * wxyz_ref[2:3, :]
    b = b * gamma_ref[...] - beta_ref[...]
    o_ref[...] = jnp.maximum(s_ref[...] - b, 0.0)


def _run_epilogue(s, pif32, wxyz, gamma_row, beta_row, mw):
    m_pad = s.shape[0]
    return pl.pallas_call(
        _epi_body,
        grid=(m_pad // mw,),
        in_specs=[
            pl.BlockSpec((mw, C_OUT), lambda i: (i, 0)),
            pl.BlockSpec((mw, 3), lambda i: (i, 0)),
            pl.BlockSpec((8, C_OUT), lambda i: (0, 0)),
            pl.BlockSpec((1, C_OUT), lambda i: (0, 0)),
            pl.BlockSpec((1, C_OUT), lambda i: (0, 0)),
        ],
        out_specs=pl.BlockSpec((mw, C_OUT), lambda i: (i, 0)),
        out_shape=jax.ShapeDtypeStruct((m_pad, C_OUT), jnp.float32),
    )(s, pif32, wxyz, gamma_row, beta_row)


def kernel(xyz, xyz_batch_cnt, point_features, pillar_indices,
           pillar_set_indices, point_set_indices, W1, gamma1, beta1):
    n = point_features.shape[0]
    m = pillar_indices.shape[0]
    l = pillar_set_indices.shape[0]
    mw = (-(-m // NW) + 7) // 8 * 8
    m_pad = NW * mw
    l_pad = -(-(l + SUP + 8) // SUP) * SUP

    w1t = W1.T
    gamma_row = gamma1.reshape(1, C_OUT)
    beta_row = beta1.reshape(1, C_OUT)

    g = jnp.concatenate([point_features, xyz], axis=1)
    a = _run_mlp(g, w1t, gamma_row)

    thresholds = jnp.arange(NW + 1, dtype=jnp.int32) * mw
    bounds = jnp.searchsorted(pillar_set_indices, thresholds,
                              side="left").astype(jnp.int32)
    bounds_pad = jnp.zeros((48,), jnp.int32).at[: NW + 1].set(bounds)

    psi_pad = jnp.full((l_pad,), jnp.int32(1 << 29), jnp.int32)
    psi_pad = psi_pad.at[:l].set(pillar_set_indices)
    pidx_pad = jnp.zeros((l_pad,), jnp.int32).at[:l].set(point_set_indices)

    s = _make_segmax(n, l_pad, mw, m_pad)(a, pidx_pad, psi_pad, bounds_pad)

    pif32 = jnp.zeros((m_pad, 3), jnp.float32)
    pif32 = pif32.at[:m].set(pillar_indices.astype(jnp.float32))
    wxyz = jnp.zeros((8, C_OUT), jnp.float32).at[:3].set(w1t[29:32])
    out = _run_epilogue(s, pif32, wxyz, gamma_row, beta_row, mw)
    return out[:m]
